# Initial kernel scaffold; baseline (speedup 1.0000x reference)
import jax
import jax.numpy as jnp
from jax.experimental import pallas as pl


def kernel(tokens):
    raise NotImplementedError("write your pallas kernel here")


# speedup vs baseline: 10.4824x; 10.4824x over previous
import functools

import jax
import jax.numpy as jnp
from jax.experimental import pallas as pl

TOPK = 8
SEASON = (1, 24)
NEG_INF = float("-inf")


def _normalize_body(t_ref, o_ref):
    x = t_ref[0]
    n2 = jnp.sum(x * x, axis=-1, keepdims=True)
    norm = jnp.sqrt(n2)
    o_ref[0] = x / jnp.maximum(norm, 1e-12)


def _mask_body(xq_ref, xk_ref, o_ref, *, tm, l, topk, season):
    q = xq_ref[0]
    k = xk_ref[0]
    sim = jax.lax.dot_general(
        q, k, (((1,), (1,)), ((), ())), preferred_element_type=jnp.float32)

    col = jax.lax.broadcasted_iota(jnp.int32, (tm, l), 1)
    simw = sim
    sel = jnp.zeros((tm, l), dtype=jnp.bool_)
    for _ in range(topk):
        m = jnp.max(simw, axis=1, keepdims=True)
        hit = simw == m
        cand = jnp.where(hit, col, l)
        j0 = jnp.min(cand, axis=1, keepdims=True)
        first = col == j0
        sel = jnp.logical_or(sel, first)
        simw = jnp.where(first, NEG_INF, simw)

    i = pl.program_id(1)
    row = i * tm + jax.lax.broadcasted_iota(jnp.int32, (tm, l), 0)
    diff = row - col
    keep = sel | (diff == 0)
    for d in season:
        keep = keep | (diff == d) | (diff == -d)
    o_ref[0, 0] = jnp.where(keep, 0.0, NEG_INF).astype(jnp.float32)


@jax.jit
def kernel(tokens):
    b, l, d = tokens.shape
    tm = 256

    xn = pl.pallas_call(
        _normalize_body,
        grid=(b, l // tm),
        in_specs=[pl.BlockSpec((1, tm, d), lambda bi, i: (bi, i, 0))],
        out_specs=pl.BlockSpec((1, tm, d), lambda bi, i: (bi, i, 0)),
        out_shape=jax.ShapeDtypeStruct((b, l, d), jnp.float32),
    )(tokens)

    body = functools.partial(_mask_body, tm=tm, l=l, topk=min(TOPK, l),
                             season=SEASON)
    out = pl.pallas_call(
        body,
        grid=(b, l // tm),
        in_specs=[
            pl.BlockSpec((1, tm, d), lambda bi, i: (bi, i, 0)),
            pl.BlockSpec((1, l, d), lambda bi, i: (bi, 0, 0)),
        ],
        out_specs=pl.BlockSpec((1, 1, tm, l), lambda bi, i: (bi, 0, i, 0)),
        out_shape=jax.ShapeDtypeStruct((b, 1, l, l), jnp.float32),
    )(xn, xn)
    return out
